# baseline (speedup 1.0000x reference)
import functools

import jax
import jax.numpy as jnp
from jax import lax
from jax.experimental import pallas as pl
from jax.experimental.pallas import tpu as pltpu
from jax.experimental.pallas import tpu_sc as plsc

N = 10000
E = 320000
D_IN = 128
H = 64

NC = 2
NS = 16
NW = NC * NS
CLEN = 128
CHUNKS = 80
NBUF = 2
CPC = 8
GROUPS = CHUNKS // CPC
EPT = CHUNKS * CLEN
EPAD = NW * EPT
NPAD = 10240
RPT = NPAD // NS

_mesh = plsc.VectorSubcoreMesh(core_axis_name="c", subcore_axis_name="s")


@functools.partial(
    pl.kernel,
    out_type=jax.ShapeDtypeStruct((NC, NPAD), jnp.float32),
    mesh=_mesh,
    compiler_params=pltpu.CompilerParams(needs_layout_passes=False),
    scratch_types=[
        pltpu.VMEM((GROUPS, CPC * CLEN), jnp.int32),
        pltpu.VMEM((NPAD,), jnp.float32),
        pltpu.VMEM((NS, RPT), jnp.float32),
        pltpu.VMEM((RPT,), jnp.float32),
        pltpu.VMEM_SHARED((NS, NPAD), jnp.float32),
    ],
)
def _deg_kernel(dst_hbm, deg_out, dst_v, hist_v, red_v, out_v, deg_sh):
    c = lax.axis_index("c")
    s = lax.axis_index("s")
    wid = s * NC + c

    zeros16 = jnp.zeros((16,), jnp.float32)
    ones16 = jnp.ones((16,), jnp.float32)

    def _zero(i, carry):
        hist_v[pl.ds(i * 16, 16)] = zeros16
        return carry

    lax.fori_loop(0, NPAD // 16, _zero, 0)

    pltpu.sync_copy(dst_hbm.at[wid], dst_v)

    def _hist(ch, carry):
        for g in range(CPC * CLEN // 16):
            idx = dst_v[ch, pl.ds(g * 16, 16)]
            plsc.addupdate_scatter(hist_v, [idx], ones16)
        return carry

    lax.fori_loop(0, GROUPS, _hist, 0)

    pltpu.sync_copy(hist_v, deg_sh.at[s])
    plsc.subcore_barrier()

    for r in range(NS):
        pltpu.sync_copy(deg_sh.at[r, pl.ds(s * RPT, RPT)], red_v.at[r])

    def _red(i, carry):
        acc = red_v[0, pl.ds(i * 16, 16)]
        for r in range(1, NS):
            acc = acc + red_v[r, pl.ds(i * 16, 16)]
        out_v[pl.ds(i * 16, 16)] = acc
        return carry

    lax.fori_loop(0, RPT // 16, _red, 0)

    pltpu.sync_copy(out_v, deg_out.at[c, pl.ds(s * RPT, RPT)])


@functools.partial(
    pl.kernel,
    out_type=jax.ShapeDtypeStruct((NC, NPAD, H), jnp.bfloat16),
    mesh=_mesh,
    compiler_params=pltpu.CompilerParams(
        needs_layout_passes=False, use_tc_tiling_on_sc=False),
    scratch_types=[
        pltpu.VMEM((GROUPS, CPC * CLEN), jnp.int32),
        pltpu.VMEM((GROUPS, CPC * CLEN), jnp.int32),
        [pltpu.VMEM((CPC * CLEN, H), jnp.bfloat16)] * NBUF,
        pltpu.VMEM_SHARED((NPAD, H), jnp.bfloat16),
        [pltpu.SemaphoreType.DMA] * NBUF,
    ],
)
def _msg_kernel(y_hbm, src_hbm, dst_hbm, acc_out,
                src_v, dst_v, bufs, acc_sh, gsems):
    c = lax.axis_index("c")
    s = lax.axis_index("s")
    wid = s * NC + c

    zeros32 = jnp.zeros((32,), jnp.bfloat16)

    def _zero(i, carry):
        for j in range(H // 32):
            bufs[0][i, pl.ds(j * 32, 32)] = zeros32
        return carry

    lax.fori_loop(0, CLEN, _zero, 0)
    for k in range(RPT // CLEN):
        pltpu.sync_copy(bufs[0].at[pl.ds(0, CLEN)],
                        acc_sh.at[pl.ds(s * RPT + k * CLEN, CLEN)])
    plsc.subcore_barrier()

    pltpu.sync_copy(src_hbm.at[wid], src_v)
    pltpu.sync_copy(dst_hbm.at[wid], dst_v)

    def _gather(g, k):
        pltpu.async_copy(y_hbm.at[src_v.at[g]], bufs[k], gsems[k])

    def _wait_gather(g, k):
        pltpu.make_async_copy(y_hbm.at[src_v.at[g]], bufs[k], gsems[k]).wait()

    _gather(0, 0)

    def _body(i, carry):
        for par in range(2):
            g = 2 * i + par
            k, kk = (par, 1 - par)

            @pl.when(g + 1 < GROUPS)
            def _():
                _gather(g + 1, kk)

            _wait_gather(g, k)
            pltpu.sync_copy(bufs[k], acc_sh.at[dst_v.at[g]], add=True)
        return carry

    lax.fori_loop(0, GROUPS // 2, _body, 0)
    plsc.subcore_barrier()

    pltpu.sync_copy(acc_sh.at[pl.ds(s * RPT, RPT)],
                    acc_out.at[c, pl.ds(s * RPT, RPT)])


def _tc1_body(x_ref, w_ref, d_ref, y_ref, dinv_ref):
    deg = d_ref[0] + d_ref[1] + 1.0
    dinv = lax.rsqrt(deg)
    xw = jnp.dot(x_ref[...], w_ref[...], preferred_element_type=jnp.float32)
    y_ref[...] = (xw * dinv).astype(jnp.bfloat16)
    dinv_ref[...] = dinv


_BLK1 = 1024


def _tc1(x_pad, W, deg3):
    return pl.pallas_call(
        _tc1_body,
        grid=(NPAD // _BLK1,),
        in_specs=[
            pl.BlockSpec((_BLK1, D_IN), lambda i: (i, 0)),
            pl.BlockSpec((D_IN, H), lambda i: (0, 0)),
            pl.BlockSpec((NC, _BLK1, 1), lambda i: (0, i, 0)),
        ],
        out_specs=[
            pl.BlockSpec((_BLK1, H), lambda i: (i, 0)),
            pl.BlockSpec((_BLK1, 1), lambda i: (i, 0)),
        ],
        out_shape=[
            jax.ShapeDtypeStruct((NPAD, H), jnp.bfloat16),
            jax.ShapeDtypeStruct((NPAD, 1), jnp.float32),
        ],
    )(x_pad, W, deg3)


def _tc2_body(acc_ref, y_ref, dinv_ref, h_ref, b_ref,
              wri, wzi, wni, wrh, wzh, wnh,
              bri, bzi, bni, brh, bzh, bnh, o_ref):
    dinv = dinv_ref[...]
    f32sum = (acc_ref[0].astype(jnp.float32) + acc_ref[1].astype(jnp.float32)
              + y_ref[...].astype(jnp.float32))
    gcn = dinv * f32sum + b_ref[...]
    h = h_ref[...]
    f32 = jnp.float32
    i_r = jnp.dot(gcn, wri[...], preferred_element_type=f32) + bri[...]
    i_z = jnp.dot(gcn, wzi[...], preferred_element_type=f32) + bzi[...]
    i_n = jnp.dot(gcn, wni[...], preferred_element_type=f32) + bni[...]
    h_r = jnp.dot(h, wrh[...], preferred_element_type=f32) + brh[...]
    h_z = jnp.dot(h, wzh[...], preferred_element_type=f32) + bzh[...]
    h_n = jnp.dot(h, wnh[...], preferred_element_type=f32) + bnh[...]
    r = jax.nn.sigmoid(i_r + h_r)
    z = jax.nn.sigmoid(i_z + h_z)
    n = jnp.tanh(i_n + r * h_n)
    o_ref[...] = (1.0 - z) * n + z * h


_BLK2 = 1000


def _tc2(acc, y, dinv, h_mem, b, ws, bs):
    row = lambda i: (i, 0)
    full = lambda i: (0, 0)
    return pl.pallas_call(
        _tc2_body,
        grid=(N // _BLK2,),
        in_specs=[
            pl.BlockSpec((NC, _BLK2, H), lambda i: (0, i, 0)),
            pl.BlockSpec((_BLK2, H), row),
            pl.BlockSpec((_BLK2, 1), row),
            pl.BlockSpec((_BLK2, H), row),
            pl.BlockSpec((1, H), full),
        ]
        + [pl.BlockSpec((H, H), full)] * 6
        + [pl.BlockSpec((1, H), full)] * 6,
        out_specs=pl.BlockSpec((_BLK2, H), row),
        out_shape=jax.ShapeDtypeStruct((N, H), jnp.float32),
    )(acc, y, dinv, h_mem, b, *ws, *bs)


def kernel(x, edge_index, W, b, W_ih, W_hh, b_ih, b_hh, h_mem):
    pad_row = N + jnp.arange(EPAD - E, dtype=jnp.int32) % (NPAD - N)
    srcp = jnp.concatenate(
        [edge_index[0].astype(jnp.int32), pad_row]
    ).reshape(NW, GROUPS, CPC * CLEN)
    dstp = jnp.concatenate(
        [edge_index[1].astype(jnp.int32), pad_row]
    ).reshape(NW, GROUPS, CPC * CLEN)

    deg2 = _deg_kernel(dstp)
    deg3 = deg2[:, :, None]

    x_pad = jnp.concatenate(
        [x, jnp.zeros((NPAD - N, D_IN), jnp.float32)])
    y, dinv = _tc1(x_pad, W, deg3)

    acc = _msg_kernel(y, srcp, dstp)

    WiT = W_ih.T
    WhT = W_hh.T
    ws = (WiT[:, :H], WiT[:, H:2 * H], WiT[:, 2 * H:],
          WhT[:, :H], WhT[:, H:2 * H], WhT[:, 2 * H:])
    bs = (b_ih[None, :H], b_ih[None, H:2 * H], b_ih[None, 2 * H:],
          b_hh[None, :H], b_hh[None, H:2 * H], b_hh[None, 2 * H:])

    return _tc2(acc, y, dinv, h_mem, b[None, :], ws, bs)


# speedup vs baseline: 1.0046x; 1.0046x over previous
import functools

import jax
import jax.numpy as jnp
from jax import lax
from jax.experimental import pallas as pl
from jax.experimental.pallas import tpu as pltpu
from jax.experimental.pallas import tpu_sc as plsc

N = 10000
E = 320000
D_IN = 128
H = 64

NC = 2
NS = 16
NW = NC * NS
CLEN = 128
CHUNKS = 80
NBUF = 2
CPC = 4
GROUPS = CHUNKS // CPC
EPT = CHUNKS * CLEN
EPAD = NW * EPT
NPAD = 10240
RPT = NPAD // NS

_mesh = plsc.VectorSubcoreMesh(core_axis_name="c", subcore_axis_name="s")


@functools.partial(
    pl.kernel,
    out_type=jax.ShapeDtypeStruct((NC, NPAD), jnp.float32),
    mesh=_mesh,
    compiler_params=pltpu.CompilerParams(needs_layout_passes=False),
    scratch_types=[
        pltpu.VMEM((GROUPS, CPC * CLEN), jnp.int32),
        pltpu.VMEM((NPAD,), jnp.float32),
        pltpu.VMEM((NS, RPT), jnp.float32),
        pltpu.VMEM((RPT,), jnp.float32),
        pltpu.VMEM_SHARED((NS, NPAD), jnp.float32),
    ],
)
def _deg_kernel(dst_hbm, deg_out, dst_v, hist_v, red_v, out_v, deg_sh):
    c = lax.axis_index("c")
    s = lax.axis_index("s")
    wid = s * NC + c

    zeros16 = jnp.zeros((16,), jnp.float32)
    ones16 = jnp.ones((16,), jnp.float32)

    def _zero(i, carry):
        hist_v[pl.ds(i * 16, 16)] = zeros16
        return carry

    lax.fori_loop(0, NPAD // 16, _zero, 0)

    pltpu.sync_copy(dst_hbm.at[wid], dst_v)

    def _hist(ch, carry):
        for g in range(CPC * CLEN // 16):
            idx = dst_v[ch, pl.ds(g * 16, 16)]
            plsc.addupdate_scatter(hist_v, [idx], ones16)
        return carry

    lax.fori_loop(0, GROUPS, _hist, 0)

    pltpu.sync_copy(hist_v, deg_sh.at[s])
    plsc.subcore_barrier()

    for r in range(NS):
        pltpu.sync_copy(deg_sh.at[r, pl.ds(s * RPT, RPT)], red_v.at[r])

    def _red(i, carry):
        acc = red_v[0, pl.ds(i * 16, 16)]
        for r in range(1, NS):
            acc = acc + red_v[r, pl.ds(i * 16, 16)]
        out_v[pl.ds(i * 16, 16)] = acc
        return carry

    lax.fori_loop(0, RPT // 16, _red, 0)

    pltpu.sync_copy(out_v, deg_out.at[c, pl.ds(s * RPT, RPT)])


@functools.partial(
    pl.kernel,
    out_type=jax.ShapeDtypeStruct((NC, NPAD, H), jnp.bfloat16),
    mesh=_mesh,
    compiler_params=pltpu.CompilerParams(
        needs_layout_passes=False, use_tc_tiling_on_sc=False),
    scratch_types=[
        pltpu.VMEM((GROUPS, CPC * CLEN), jnp.int32),
        pltpu.VMEM((GROUPS, CPC * CLEN), jnp.int32),
        [pltpu.VMEM((CPC * CLEN, H), jnp.bfloat16)] * NBUF,
        pltpu.VMEM_SHARED((NPAD, H), jnp.bfloat16),
        [pltpu.SemaphoreType.DMA] * NBUF,
    ],
)
def _msg_kernel(y_hbm, src_hbm, dst_hbm, acc_out,
                src_v, dst_v, bufs, acc_sh, gsems):
    c = lax.axis_index("c")
    s = lax.axis_index("s")
    wid = s * NC + c

    zeros32 = jnp.zeros((32,), jnp.bfloat16)

    def _zero(i, carry):
        for j in range(H // 32):
            bufs[0][i, pl.ds(j * 32, 32)] = zeros32
        return carry

    lax.fori_loop(0, CLEN, _zero, 0)
    for k in range(RPT // CLEN):
        pltpu.sync_copy(bufs[0].at[pl.ds(0, CLEN)],
                        acc_sh.at[pl.ds(s * RPT + k * CLEN, CLEN)])
    plsc.subcore_barrier()

    pltpu.sync_copy(src_hbm.at[wid], src_v)
    pltpu.sync_copy(dst_hbm.at[wid], dst_v)

    def _gather(g, k):
        pltpu.async_copy(y_hbm.at[src_v.at[g]], bufs[k], gsems[k])

    def _wait_gather(g, k):
        pltpu.make_async_copy(y_hbm.at[src_v.at[g]], bufs[k], gsems[k]).wait()

    _gather(0, 0)

    def _body(i, carry):
        for par in range(2):
            g = 2 * i + par
            k, kk = (par, 1 - par)

            @pl.when(g + 1 < GROUPS)
            def _():
                _gather(g + 1, kk)

            _wait_gather(g, k)
            pltpu.sync_copy(bufs[k], acc_sh.at[dst_v.at[g]], add=True)
        return carry

    lax.fori_loop(0, GROUPS // 2, _body, 0)
    plsc.subcore_barrier()

    pltpu.sync_copy(acc_sh.at[pl.ds(s * RPT, RPT)],
                    acc_out.at[c, pl.ds(s * RPT, RPT)])


def _tc1_body(x_ref, w_ref, d_ref, y_ref, dinv_ref):
    deg = d_ref[0] + d_ref[1] + 1.0
    dinv = lax.rsqrt(deg)
    xw = jnp.dot(x_ref[...], w_ref[...], preferred_element_type=jnp.float32)
    y_ref[...] = (xw * dinv).astype(jnp.bfloat16)
    dinv_ref[...] = dinv


_BLK1 = 1024


def _tc1(x_pad, W, deg3):
    return pl.pallas_call(
        _tc1_body,
        grid=(NPAD // _BLK1,),
        in_specs=[
            pl.BlockSpec((_BLK1, D_IN), lambda i: (i, 0)),
            pl.BlockSpec((D_IN, H), lambda i: (0, 0)),
            pl.BlockSpec((NC, _BLK1, 1), lambda i: (0, i, 0)),
        ],
        out_specs=[
            pl.BlockSpec((_BLK1, H), lambda i: (i, 0)),
            pl.BlockSpec((_BLK1, 1), lambda i: (i, 0)),
        ],
        out_shape=[
            jax.ShapeDtypeStruct((NPAD, H), jnp.bfloat16),
            jax.ShapeDtypeStruct((NPAD, 1), jnp.float32),
        ],
    )(x_pad, W, deg3)


def _tc2_body(acc_ref, y_ref, dinv_ref, h_ref, b_ref,
              wri, wzi, wni, wrh, wzh, wnh,
              bri, bzi, bni, brh, bzh, bnh, o_ref):
    dinv = dinv_ref[...]
    f32sum = (acc_ref[0].astype(jnp.float32) + acc_ref[1].astype(jnp.float32)
              + y_ref[...].astype(jnp.float32))
    gcn = dinv * f32sum + b_ref[...]
    h = h_ref[...]
    f32 = jnp.float32
    i_r = jnp.dot(gcn, wri[...], preferred_element_type=f32) + bri[...]
    i_z = jnp.dot(gcn, wzi[...], preferred_element_type=f32) + bzi[...]
    i_n = jnp.dot(gcn, wni[...], preferred_element_type=f32) + bni[...]
    h_r = jnp.dot(h, wrh[...], preferred_element_type=f32) + brh[...]
    h_z = jnp.dot(h, wzh[...], preferred_element_type=f32) + bzh[...]
    h_n = jnp.dot(h, wnh[...], preferred_element_type=f32) + bnh[...]
    r = jax.nn.sigmoid(i_r + h_r)
    z = jax.nn.sigmoid(i_z + h_z)
    n = jnp.tanh(i_n + r * h_n)
    o_ref[...] = (1.0 - z) * n + z * h


_BLK2 = 1000


def _tc2(acc, y, dinv, h_mem, b, ws, bs):
    row = lambda i: (i, 0)
    full = lambda i: (0, 0)
    return pl.pallas_call(
        _tc2_body,
        grid=(N // _BLK2,),
        in_specs=[
            pl.BlockSpec((NC, _BLK2, H), lambda i: (0, i, 0)),
            pl.BlockSpec((_BLK2, H), row),
            pl.BlockSpec((_BLK2, 1), row),
            pl.BlockSpec((_BLK2, H), row),
            pl.BlockSpec((1, H), full),
        ]
        + [pl.BlockSpec((H, H), full)] * 6
        + [pl.BlockSpec((1, H), full)] * 6,
        out_specs=pl.BlockSpec((_BLK2, H), row),
        out_shape=jax.ShapeDtypeStruct((N, H), jnp.float32),
    )(acc, y, dinv, h_mem, b, *ws, *bs)


def kernel(x, edge_index, W, b, W_ih, W_hh, b_ih, b_hh, h_mem):
    pad_row = N + jnp.arange(EPAD - E, dtype=jnp.int32) % (NPAD - N)
    srcp = jnp.concatenate(
        [edge_index[0].astype(jnp.int32), pad_row]
    ).reshape(NW, GROUPS, CPC * CLEN)
    dstp = jnp.concatenate(
        [edge_index[1].astype(jnp.int32), pad_row]
    ).reshape(NW, GROUPS, CPC * CLEN)

    deg2 = _deg_kernel(dstp)
    deg3 = deg2[:, :, None]

    x_pad = jnp.concatenate(
        [x, jnp.zeros((NPAD - N, D_IN), jnp.float32)])
    y, dinv = _tc1(x_pad, W, deg3)

    acc = _msg_kernel(y, srcp, dstp)

    WiT = W_ih.T
    WhT = W_hh.T
    ws = (WiT[:, :H], WiT[:, H:2 * H], WiT[:, 2 * H:],
          WhT[:, :H], WhT[:, H:2 * H], WhT[:, 2 * H:])
    bs = (b_ih[None, :H], b_ih[None, H:2 * H], b_ih[None, 2 * H:],
          b_hh[None, :H], b_hh[None, H:2 * H], b_hh[None, 2 * H:])

    return _tc2(acc, y, dinv, h_mem, b[None, :], ws, bs)


# speedup vs baseline: 1.0249x; 1.0202x over previous
import functools

import jax
import jax.numpy as jnp
from jax import lax
from jax.experimental import pallas as pl
from jax.experimental.pallas import tpu as pltpu
from jax.experimental.pallas import tpu_sc as plsc

N = 10000
E = 320000
D_IN = 128
H = 64

NC = 2
NS = 16
NW = NC * NS
EPT = E // NW
GROUPS = 20
GLEN = EPT // GROUPS
NPAD = 10240
RPT = NPAD // NS

_mesh = plsc.VectorSubcoreMesh(core_axis_name="c", subcore_axis_name="s")


@functools.partial(
    pl.kernel,
    out_type=jax.ShapeDtypeStruct((NC, NPAD), jnp.float32),
    mesh=_mesh,
    compiler_params=pltpu.CompilerParams(needs_layout_passes=False),
    scratch_types=[
        pltpu.VMEM((EPT,), jnp.int32),
        pltpu.VMEM((NPAD,), jnp.float32),
        pltpu.VMEM((NS, RPT), jnp.float32),
        pltpu.VMEM((RPT,), jnp.float32),
        pltpu.VMEM_SHARED((NS, NPAD), jnp.float32),
    ],
)
def _deg_kernel(dst_hbm, deg_out, dst_v, hist_v, red_v, out_v, deg_sh):
    c = lax.axis_index("c")
    s = lax.axis_index("s")
    wid = s * NC + c

    zeros16 = jnp.zeros((16,), jnp.float32)
    ones16 = jnp.ones((16,), jnp.float32)

    def _zero(i, carry):
        hist_v[pl.ds(i * 16, 16)] = zeros16
        return carry

    lax.fori_loop(0, NPAD // 16, _zero, 0)

    pltpu.sync_copy(dst_hbm.at[wid], dst_v)

    def _hist(i, carry):
        for g in range(5):
            idx = dst_v[pl.ds((i * 5 + g) * 16, 16)]
            plsc.addupdate_scatter(hist_v, [idx], ones16)
        return carry

    lax.fori_loop(0, EPT // 80, _hist, 0)

    pltpu.sync_copy(hist_v, deg_sh.at[s])
    plsc.subcore_barrier()

    for r in range(NS):
        pltpu.sync_copy(deg_sh.at[r, pl.ds(s * RPT, RPT)], red_v.at[r])

    def _red(i, carry):
        acc = red_v[0, pl.ds(i * 16, 16)]
        for r in range(1, NS):
            acc = acc + red_v[r, pl.ds(i * 16, 16)]
        out_v[pl.ds(i * 16, 16)] = acc
        return carry

    lax.fori_loop(0, RPT // 16, _red, 0)

    pltpu.sync_copy(out_v, deg_out.at[c, pl.ds(s * RPT, RPT)])


@functools.partial(
    pl.kernel,
    out_type=jax.ShapeDtypeStruct((NC, NPAD, H), jnp.bfloat16),
    mesh=_mesh,
    compiler_params=pltpu.CompilerParams(
        needs_layout_passes=False, use_tc_tiling_on_sc=False),
    scratch_types=[
        pltpu.VMEM((GROUPS, GLEN), jnp.int32),
        pltpu.VMEM((GROUPS, GLEN), jnp.int32),
        [pltpu.VMEM((GLEN, H), jnp.bfloat16)] * 2,
        pltpu.VMEM_SHARED((NPAD, H), jnp.bfloat16),
        [pltpu.SemaphoreType.DMA] * 2,
    ],
)
def _msg_kernel(y_hbm, src_hbm, dst_hbm, acc_out,
                src_v, dst_v, bufs, acc_sh, gsems):
    c = lax.axis_index("c")
    s = lax.axis_index("s")
    wid = s * NC + c

    zeros32 = jnp.zeros((32,), jnp.bfloat16)

    def _zero(i, carry):
        for j in range(H // 32):
            bufs[0][i, pl.ds(j * 32, 32)] = zeros32
        return carry

    lax.fori_loop(0, RPT // 2, _zero, 0)
    for k in range(2):
        pltpu.sync_copy(bufs[0].at[pl.ds(0, RPT // 2)],
                        acc_sh.at[pl.ds(s * RPT + k * (RPT // 2), RPT // 2)])
    plsc.subcore_barrier()

    pltpu.sync_copy(src_hbm.at[wid], src_v)
    pltpu.sync_copy(dst_hbm.at[wid], dst_v)

    def _gather(g, k):
        pltpu.async_copy(y_hbm.at[src_v.at[g]], bufs[k], gsems[k])

    def _wait_gather(g, k):
        pltpu.make_async_copy(y_hbm.at[src_v.at[g]], bufs[k], gsems[k]).wait()

    _gather(0, 0)

    def _body(i, carry):
        for par in range(2):
            g = 2 * i + par
            k, kk = (par, 1 - par)

            @pl.when(g + 1 < GROUPS)
            def _():
                _gather(g + 1, kk)

            _wait_gather(g, k)
            pltpu.sync_copy(bufs[k], acc_sh.at[dst_v.at[g]], add=True)
        return carry

    lax.fori_loop(0, GROUPS // 2, _body, 0)
    plsc.subcore_barrier()

    pltpu.sync_copy(acc_sh.at[pl.ds(s * RPT, RPT)],
                    acc_out.at[c, pl.ds(s * RPT, RPT)])


def _tc1_body(x_ref, w_ref, d_ref, y_ref, dinv_ref):
    deg = d_ref[0] + d_ref[1] + 1.0
    dinv = lax.rsqrt(deg)
    xw = jnp.dot(x_ref[...], w_ref[...], preferred_element_type=jnp.float32)
    y_ref[...] = (xw * dinv).astype(jnp.bfloat16)
    dinv_ref[...] = dinv


_BLK1 = 1000


def _tc1(x, W, deg3):
    return pl.pallas_call(
        _tc1_body,
        grid=(N // _BLK1,),
        in_specs=[
            pl.BlockSpec((_BLK1, D_IN), lambda i: (i, 0)),
            pl.BlockSpec((D_IN, H), lambda i: (0, 0)),
            pl.BlockSpec((NC, _BLK1, 1), lambda i: (0, i, 0)),
        ],
        out_specs=[
            pl.BlockSpec((_BLK1, H), lambda i: (i, 0)),
            pl.BlockSpec((_BLK1, 1), lambda i: (i, 0)),
        ],
        out_shape=[
            jax.ShapeDtypeStruct((N, H), jnp.bfloat16),
            jax.ShapeDtypeStruct((N, 1), jnp.float32),
        ],
    )(x, W, deg3)


def _tc2_body(acc_ref, y_ref, dinv_ref, h_ref, b_ref,
              wri, wzi, wni, wrh, wzh, wnh,
              bri, bzi, bni, brh, bzh, bnh, o_ref):
    dinv = dinv_ref[...]
    f32sum = (acc_ref[0].astype(jnp.float32) + acc_ref[1].astype(jnp.float32)
              + y_ref[...].astype(jnp.float32))
    gcn = dinv * f32sum + b_ref[...]
    h = h_ref[...]
    f32 = jnp.float32
    i_r = jnp.dot(gcn, wri[...], preferred_element_type=f32) + bri[...]
    i_z = jnp.dot(gcn, wzi[...], preferred_element_type=f32) + bzi[...]
    i_n = jnp.dot(gcn, wni[...], preferred_element_type=f32) + bni[...]
    h_r = jnp.dot(h, wrh[...], preferred_element_type=f32) + brh[...]
    h_z = jnp.dot(h, wzh[...], preferred_element_type=f32) + bzh[...]
    h_n = jnp.dot(h, wnh[...], preferred_element_type=f32) + bnh[...]
    r = jax.nn.sigmoid(i_r + h_r)
    z = jax.nn.sigmoid(i_z + h_z)
    n = jnp.tanh(i_n + r * h_n)
    o_ref[...] = (1.0 - z) * n + z * h


_BLK2 = 1000


def _tc2(acc, y, dinv, h_mem, b, ws, bs):
    row = lambda i: (i, 0)
    full = lambda i: (0, 0)
    return pl.pallas_call(
        _tc2_body,
        grid=(N // _BLK2,),
        in_specs=[
            pl.BlockSpec((NC, _BLK2, H), lambda i: (0, i, 0)),
            pl.BlockSpec((_BLK2, H), row),
            pl.BlockSpec((_BLK2, 1), row),
            pl.BlockSpec((_BLK2, H), row),
            pl.BlockSpec((1, H), full),
        ]
        + [pl.BlockSpec((H, H), full)] * 6
        + [pl.BlockSpec((1, H), full)] * 6,
        out_specs=pl.BlockSpec((_BLK2, H), row),
        out_shape=jax.ShapeDtypeStruct((N, H), jnp.float32),
    )(acc, y, dinv, h_mem, b, *ws, *bs)


def kernel(x, edge_index, W, b, W_ih, W_hh, b_ih, b_hh, h_mem):
    ei = edge_index.astype(jnp.int32)
    srcp = ei[0].reshape(NW, GROUPS, GLEN)
    dstp = ei[1].reshape(NW, GROUPS, GLEN)
    dstf = ei[1].reshape(NW, EPT)

    deg2 = _deg_kernel(dstf)
    deg3 = deg2[:, :, None]

    y, dinv = _tc1(x, W, deg3)

    acc = _msg_kernel(y, srcp, dstp)

    WiT = W_ih.T
    WhT = W_hh.T
    ws = (WiT[:, :H], WiT[:, H:2 * H], WiT[:, 2 * H:],
          WhT[:, :H], WhT[:, H:2 * H], WhT[:, 2 * H:])
    bs = (b_ih[None, :H], b_ih[None, H:2 * H], b_ih[None, 2 * H:],
          b_hh[None, :H], b_hh[None, H:2 * H], b_hh[None, 2 * H:])

    return _tc2(acc, y, dinv, h_mem, b[None, :], ws, bs)


# speedup vs baseline: 1.0263x; 1.0014x over previous
import functools

import jax
import jax.numpy as jnp
from jax import lax
from jax.experimental import pallas as pl
from jax.experimental.pallas import tpu as pltpu
from jax.experimental.pallas import tpu_sc as plsc

N = 10000
E = 320000
D_IN = 128
H = 64

NC = 2
NS = 16
NW = NC * NS
EPT = E // NW
GROUPS = 20
GLEN = EPT // GROUPS
NPAD = 10240
RPT = NPAD // NS

_mesh = plsc.VectorSubcoreMesh(core_axis_name="c", subcore_axis_name="s")


@functools.partial(
    pl.kernel,
    out_type=jax.ShapeDtypeStruct((NC, NPAD), jnp.float32),
    mesh=_mesh,
    compiler_params=pltpu.CompilerParams(needs_layout_passes=False),
    scratch_types=[
        pltpu.VMEM((EPT,), jnp.int32),
        pltpu.VMEM((NPAD,), jnp.float32),
        pltpu.VMEM((NS, RPT), jnp.float32),
        pltpu.VMEM((RPT,), jnp.float32),
        pltpu.VMEM_SHARED((NS, NPAD), jnp.float32),
    ],
)
def _deg_kernel(dst_hbm, deg_out, dst_v, hist_v, red_v, out_v, deg_sh):
    c = lax.axis_index("c")
    s = lax.axis_index("s")
    wid = s * NC + c

    zeros16 = jnp.zeros((16,), jnp.float32)
    ones16 = jnp.ones((16,), jnp.float32)

    def _zero(i, carry):
        hist_v[pl.ds(i * 16, 16)] = zeros16
        return carry

    lax.fori_loop(0, NPAD // 16, _zero, 0)

    pltpu.sync_copy(dst_hbm.at[wid], dst_v)

    def _hist(i, carry):
        for g in range(5):
            idx = dst_v[pl.ds((i * 5 + g) * 16, 16)]
            plsc.addupdate_scatter(hist_v, [idx], ones16)
        return carry

    lax.fori_loop(0, EPT // 80, _hist, 0)

    pltpu.sync_copy(hist_v, deg_sh.at[s])
    plsc.subcore_barrier()

    for r in range(NS):
        pltpu.sync_copy(deg_sh.at[r, pl.ds(s * RPT, RPT)], red_v.at[r])

    def _red(i, carry):
        acc = red_v[0, pl.ds(i * 16, 16)]
        for r in range(1, NS):
            acc = acc + red_v[r, pl.ds(i * 16, 16)]
        out_v[pl.ds(i * 16, 16)] = acc
        return carry

    lax.fori_loop(0, RPT // 16, _red, 0)

    pltpu.sync_copy(out_v, deg_out.at[c, pl.ds(s * RPT, RPT)])


@functools.partial(
    pl.kernel,
    out_type=jax.ShapeDtypeStruct((NC, NPAD, H), jnp.bfloat16),
    mesh=_mesh,
    compiler_params=pltpu.CompilerParams(
        needs_layout_passes=False, use_tc_tiling_on_sc=False),
    scratch_types=[
        pltpu.VMEM((GROUPS, GLEN), jnp.int32),
        pltpu.VMEM((GROUPS, GLEN), jnp.int32),
        [pltpu.VMEM((GLEN, H), jnp.bfloat16)] * 2,
        pltpu.VMEM_SHARED((NPAD, H), jnp.bfloat16),
        [pltpu.SemaphoreType.DMA] * 2,
    ],
)
def _msg_kernel(y_hbm, src_hbm, dst_hbm, acc_out,
                src_v, dst_v, bufs, acc_sh, gsems):
    c = lax.axis_index("c")
    s = lax.axis_index("s")
    wid = s * NC + c

    zeros32 = jnp.zeros((32,), jnp.bfloat16)

    def _zero(i, carry):
        for j in range(H // 32):
            bufs[0][i, pl.ds(j * 32, 32)] = zeros32
        return carry

    lax.fori_loop(0, RPT // 2, _zero, 0)
    for k in range(2):
        pltpu.sync_copy(bufs[0].at[pl.ds(0, RPT // 2)],
                        acc_sh.at[pl.ds(s * RPT + k * (RPT // 2), RPT // 2)])
    plsc.subcore_barrier()

    pltpu.sync_copy(src_hbm.at[wid], src_v)
    pltpu.sync_copy(dst_hbm.at[wid], dst_v)

    def _gather(g, k):
        pltpu.async_copy(y_hbm.at[src_v.at[g]], bufs[k], gsems[k])

    def _wait_gather(g, k):
        pltpu.make_async_copy(y_hbm.at[src_v.at[g]], bufs[k], gsems[k]).wait()

    _gather(0, 0)

    def _body(i, carry):
        for par in range(2):
            g = 2 * i + par
            k, kk = (par, 1 - par)

            @pl.when(g + 1 < GROUPS)
            def _():
                _gather(g + 1, kk)

            _wait_gather(g, k)
            pltpu.sync_copy(bufs[k], acc_sh.at[dst_v.at[g]], add=True)
        return carry

    lax.fori_loop(0, GROUPS // 2, _body, 0)
    plsc.subcore_barrier()

    pltpu.sync_copy(acc_sh.at[pl.ds(s * RPT, RPT)],
                    acc_out.at[c, pl.ds(s * RPT, RPT)])


def _tc1_body(x_ref, w_ref, d_ref, y_ref, dinv_ref):
    deg = d_ref[0] + d_ref[1] + 1.0
    dinv = lax.rsqrt(deg)
    xw = jnp.dot(x_ref[...], w_ref[...], preferred_element_type=jnp.float32)
    y_ref[...] = (xw * dinv).astype(jnp.bfloat16)
    dinv_ref[...] = dinv


_BLK1 = 1000


def _tc1(xb, Wb, deg3):
    return pl.pallas_call(
        _tc1_body,
        grid=(N // _BLK1,),
        in_specs=[
            pl.BlockSpec((_BLK1, D_IN), lambda i: (i, 0)),
            pl.BlockSpec((D_IN, H), lambda i: (0, 0)),
            pl.BlockSpec((NC, _BLK1, 1), lambda i: (0, i, 0)),
        ],
        out_specs=[
            pl.BlockSpec((_BLK1, H), lambda i: (i, 0)),
            pl.BlockSpec((_BLK1, 1), lambda i: (i, 0)),
        ],
        out_shape=[
            jax.ShapeDtypeStruct((N, H), jnp.bfloat16),
            jax.ShapeDtypeStruct((N, 1), jnp.float32),
        ],
    )(xb, Wb, deg3)


def _tc2_body(acc_ref, y_ref, dinv_ref, h_ref, b_ref,
              wri, wzi, wni, wrh, wzh, wnh,
              bri, bzi, bni, brh, bzh, bnh, o_ref):
    dinv = dinv_ref[...]
    f32sum = (acc_ref[0].astype(jnp.float32) + acc_ref[1].astype(jnp.float32)
              + y_ref[...].astype(jnp.float32))
    gcn = dinv * f32sum + b_ref[...]
    h = h_ref[...]
    f32 = jnp.float32
    i_r = jnp.dot(gcn, wri[...], preferred_element_type=f32) + bri[...]
    i_z = jnp.dot(gcn, wzi[...], preferred_element_type=f32) + bzi[...]
    i_n = jnp.dot(gcn, wni[...], preferred_element_type=f32) + bni[...]
    h_r = jnp.dot(h, wrh[...], preferred_element_type=f32) + brh[...]
    h_z = jnp.dot(h, wzh[...], preferred_element_type=f32) + bzh[...]
    h_n = jnp.dot(h, wnh[...], preferred_element_type=f32) + bnh[...]
    r = jax.nn.sigmoid(i_r + h_r)
    z = jax.nn.sigmoid(i_z + h_z)
    n = jnp.tanh(i_n + r * h_n)
    o_ref[...] = (1.0 - z) * n + z * h


_BLK2 = 1000


def _tc2(acc, y, dinv, h_mem, b, ws, bs):
    row = lambda i: (i, 0)
    full = lambda i: (0, 0)
    return pl.pallas_call(
        _tc2_body,
        grid=(N // _BLK2,),
        in_specs=[
            pl.BlockSpec((NC, _BLK2, H), lambda i: (0, i, 0)),
            pl.BlockSpec((_BLK2, H), row),
            pl.BlockSpec((_BLK2, 1), row),
            pl.BlockSpec((_BLK2, H), row),
            pl.BlockSpec((1, H), full),
        ]
        + [pl.BlockSpec((H, H), full)] * 6
        + [pl.BlockSpec((1, H), full)] * 6,
        out_specs=pl.BlockSpec((_BLK2, H), row),
        out_shape=jax.ShapeDtypeStruct((N, H), jnp.float32),
    )(acc, y, dinv, h_mem, b, *ws, *bs)


def kernel(x, edge_index, W, b, W_ih, W_hh, b_ih, b_hh, h_mem):
    ei = edge_index.astype(jnp.int32)
    srcp = ei[0].reshape(NW, GROUPS, GLEN)
    dstp = ei[1].reshape(NW, GROUPS, GLEN)
    dstf = ei[1].reshape(NW, EPT)

    deg2 = _deg_kernel(dstf)
    deg3 = deg2[:, :, None]

    y, dinv = _tc1(x.astype(jnp.bfloat16), W.astype(jnp.bfloat16),
                   deg3)

    acc = _msg_kernel(y, srcp, dstp)

    WiT = W_ih.T
    WhT = W_hh.T
    ws = (WiT[:, :H], WiT[:, H:2 * H], WiT[:, 2 * H:],
          WhT[:, :H], WhT[:, H:2 * H], WhT[:, 2 * H:])
    bs = (b_ih[None, :H], b_ih[None, H:2 * H], b_ih[None, 2 * H:],
          b_hh[None, :H], b_hh[None, H:2 * H], b_hh[None, 2 * H:])

    return _tc2(acc, y, dinv, h_mem, b[None, :], ws, bs)


# speedup vs baseline: 1.0554x; 1.0283x over previous
import functools

import jax
import jax.numpy as jnp
from jax import lax
from jax.experimental import pallas as pl
from jax.experimental.pallas import tpu as pltpu
from jax.experimental.pallas import tpu_sc as plsc

N = 10000
E = 320000
D_IN = 128
H = 64

NC = 2
NS = 16
NW = NC * NS
EPT = E // NW
GROUPS = 20
GLEN = EPT // GROUPS
NPAD = 10240
RPT = NPAD // NS

_mesh = plsc.VectorSubcoreMesh(core_axis_name="c", subcore_axis_name="s")


@functools.partial(
    pl.kernel,
    out_type=jax.ShapeDtypeStruct((NC, NPAD), jnp.float32),
    mesh=_mesh,
    compiler_params=pltpu.CompilerParams(needs_layout_passes=False),
    scratch_types=[
        pltpu.VMEM((EPT,), jnp.int32),
        pltpu.VMEM((NPAD,), jnp.float32),
        pltpu.VMEM((NS, RPT), jnp.float32),
        pltpu.VMEM((RPT,), jnp.float32),
        pltpu.VMEM_SHARED((NS, NPAD), jnp.float32),
    ],
)
def _deg_kernel(dst_hbm, deg_out, dst_v, hist_v, red_v, out_v, deg_sh):
    c = lax.axis_index("c")
    s = lax.axis_index("s")
    wid = s * NC + c

    zeros16 = jnp.zeros((16,), jnp.float32)
    ones16 = jnp.ones((16,), jnp.float32)

    def _zero(i, carry):
        hist_v[pl.ds(i * 16, 16)] = zeros16
        return carry

    lax.fori_loop(0, NPAD // 16, _zero, 0)

    pltpu.sync_copy(dst_hbm.at[wid], dst_v)

    def _hist(i, carry):
        for g in range(5):
            idx = dst_v[pl.ds((i * 5 + g) * 16, 16)]
            plsc.addupdate_scatter(hist_v, [idx], ones16)
        return carry

    lax.fori_loop(0, EPT // 80, _hist, 0)

    pltpu.sync_copy(hist_v, deg_sh.at[s])
    plsc.subcore_barrier()

    for r in range(NS):
        pltpu.sync_copy(deg_sh.at[r, pl.ds(s * RPT, RPT)], red_v.at[r])

    def _red(i, carry):
        acc = red_v[0, pl.ds(i * 16, 16)]
        for r in range(1, NS):
            acc = acc + red_v[r, pl.ds(i * 16, 16)]
        out_v[pl.ds(i * 16, 16)] = acc
        return carry

    lax.fori_loop(0, RPT // 16, _red, 0)

    pltpu.sync_copy(out_v, deg_out.at[c, pl.ds(s * RPT, RPT)])


@functools.partial(
    pl.kernel,
    out_type=jax.ShapeDtypeStruct((NC, NPAD, H), jnp.bfloat16),
    mesh=_mesh,
    compiler_params=pltpu.CompilerParams(
        needs_layout_passes=False, use_tc_tiling_on_sc=False),
    scratch_types=[
        pltpu.VMEM((GROUPS, GLEN), jnp.int32),
        pltpu.VMEM((GROUPS, GLEN), jnp.int32),
        [pltpu.VMEM((GLEN, H), jnp.bfloat16)] * 2,
        pltpu.VMEM_SHARED((NPAD, H), jnp.bfloat16),
        [pltpu.SemaphoreType.DMA] * 2,
    ],
)
def _msg_kernel(y_hbm, src_hbm, dst_hbm, acc_out,
                src_v, dst_v, bufs, acc_sh, gsems):
    c = lax.axis_index("c")
    s = lax.axis_index("s")
    wid = s * NC + c

    zeros32 = jnp.zeros((32,), jnp.bfloat16)

    def _zero(i, carry):
        for j in range(H // 32):
            bufs[0][i, pl.ds(j * 32, 32)] = zeros32
        return carry

    lax.fori_loop(0, RPT // 2, _zero, 0)
    for k in range(2):
        pltpu.sync_copy(bufs[0].at[pl.ds(0, RPT // 2)],
                        acc_sh.at[pl.ds(s * RPT + k * (RPT // 2), RPT // 2)])
    plsc.subcore_barrier()

    pltpu.sync_copy(src_hbm.at[wid], src_v)
    pltpu.sync_copy(dst_hbm.at[wid], dst_v)

    def _gather(g, k):
        pltpu.async_copy(y_hbm.at[src_v.at[g]], bufs[k], gsems[k])

    def _wait_gather(g, k):
        pltpu.make_async_copy(y_hbm.at[src_v.at[g]], bufs[k], gsems[k]).wait()

    _gather(0, 0)

    def _body(i, carry):
        for par in range(2):
            g = 2 * i + par
            k, kk = (par, 1 - par)

            @pl.when(g + 1 < GROUPS)
            def _():
                _gather(g + 1, kk)

            _wait_gather(g, k)
            pltpu.sync_copy(bufs[k], acc_sh.at[dst_v.at[g]], add=True)
        return carry

    lax.fori_loop(0, GROUPS // 2, _body, 0)
    plsc.subcore_barrier()

    pltpu.sync_copy(acc_sh.at[pl.ds(s * RPT, RPT)],
                    acc_out.at[c, pl.ds(s * RPT, RPT)])


def _tc1_body(x_ref, w_ref, d_ref, y_ref, dinv_ref):
    deg = d_ref[0] + d_ref[1] + 1.0
    dinv = lax.rsqrt(deg)
    xw = jnp.dot(x_ref[...], w_ref[...], preferred_element_type=jnp.float32)
    y_ref[...] = (xw * dinv).astype(jnp.bfloat16)
    dinv_ref[...] = dinv


_BLK1 = 1000


def _tc1(xb, Wb, deg3):
    return pl.pallas_call(
        _tc1_body,
        grid=(N // _BLK1,),
        in_specs=[
            pl.BlockSpec((_BLK1, D_IN), lambda i: (i, 0)),
            pl.BlockSpec((D_IN, H), lambda i: (0, 0)),
            pl.BlockSpec((NC, _BLK1, 1), lambda i: (0, i, 0)),
        ],
        out_specs=[
            pl.BlockSpec((_BLK1, H), lambda i: (i, 0)),
            pl.BlockSpec((_BLK1, 1), lambda i: (i, 0)),
        ],
        out_shape=[
            jax.ShapeDtypeStruct((N, H), jnp.bfloat16),
            jax.ShapeDtypeStruct((N, 1), jnp.float32),
        ],
    )(xb, Wb, deg3)


def _tc2_body(acc_ref, y_ref, dinv_ref, h_ref, b_ref,
              wri, wzi, wni, wrh, wzh, wnh,
              bri, bzi, bni, brh, bzh, bnh, o_ref):
    dinv = dinv_ref[...]
    f32sum = (acc_ref[0].astype(jnp.float32) + acc_ref[1].astype(jnp.float32)
              + y_ref[...].astype(jnp.float32))
    gcn = dinv * f32sum + b_ref[...]
    h = h_ref[...]
    f32 = jnp.float32
    i_r = jnp.dot(gcn, wri[...], preferred_element_type=f32) + bri[...]
    i_z = jnp.dot(gcn, wzi[...], preferred_element_type=f32) + bzi[...]
    i_n = jnp.dot(gcn, wni[...], preferred_element_type=f32) + bni[...]
    h_r = jnp.dot(h, wrh[...], preferred_element_type=f32) + brh[...]
    h_z = jnp.dot(h, wzh[...], preferred_element_type=f32) + bzh[...]
    h_n = jnp.dot(h, wnh[...], preferred_element_type=f32) + bnh[...]
    r = jax.nn.sigmoid(i_r + h_r)
    z = jax.nn.sigmoid(i_z + h_z)
    n = jnp.tanh(i_n + r * h_n)
    o_ref[...] = (1.0 - z) * n + z * h


_BLK2 = 2000


def _tc2(acc, y, dinv, h_mem, b, ws, bs):
    row = lambda i: (i, 0)
    full = lambda i: (0, 0)
    return pl.pallas_call(
        _tc2_body,
        grid=(N // _BLK2,),
        in_specs=[
            pl.BlockSpec((NC, _BLK2, H), lambda i: (0, i, 0)),
            pl.BlockSpec((_BLK2, H), row),
            pl.BlockSpec((_BLK2, 1), row),
            pl.BlockSpec((_BLK2, H), row),
            pl.BlockSpec((1, H), full),
        ]
        + [pl.BlockSpec((H, H), full)] * 6
        + [pl.BlockSpec((1, H), full)] * 6,
        out_specs=pl.BlockSpec((_BLK2, H), row),
        out_shape=jax.ShapeDtypeStruct((N, H), jnp.float32),
    )(acc, y, dinv, h_mem, b, *ws, *bs)


def kernel(x, edge_index, W, b, W_ih, W_hh, b_ih, b_hh, h_mem):
    ei = edge_index.astype(jnp.int32)
    srcp = ei[0].reshape(NW, GROUPS, GLEN)
    dstp = ei[1].reshape(NW, GROUPS, GLEN)
    dstf = ei[1].reshape(NW, EPT)

    deg2 = _deg_kernel(dstf)
    deg3 = deg2[:, :, None]

    y, dinv = _tc1(x.astype(jnp.bfloat16), W.astype(jnp.bfloat16),
                   deg3)

    acc = _msg_kernel(y, srcp, dstp)

    WiT = W_ih.T
    WhT = W_hh.T
    ws = (WiT[:, :H], WiT[:, H:2 * H], WiT[:, 2 * H:],
          WhT[:, :H], WhT[:, H:2 * H], WhT[:, 2 * H:])
    bs = (b_ih[None, :H], b_ih[None, H:2 * H], b_ih[None, 2 * H:],
          b_hh[None, :H], b_hh[None, H:2 * H], b_hh[None, 2 * H:])

    return _tc2(acc, y, dinv, h_mem, b[None, :], ws, bs)


# speedup vs baseline: 1.0717x; 1.0155x over previous
import functools

import jax
import jax.numpy as jnp
from jax import lax
from jax.experimental import pallas as pl
from jax.experimental.pallas import tpu as pltpu
from jax.experimental.pallas import tpu_sc as plsc

N = 10000
E = 320000
D_IN = 128
H = 64

NC = 2
NS = 16
NW = NC * NS
EPT = E // NW
GROUPS = 20
GLEN = EPT // GROUPS
NPAD = 10240
RPT = NPAD // NS

_mesh = plsc.VectorSubcoreMesh(core_axis_name="c", subcore_axis_name="s")


@functools.partial(
    pl.kernel,
    out_type=jax.ShapeDtypeStruct((NC, NPAD), jnp.float32),
    mesh=_mesh,
    compiler_params=pltpu.CompilerParams(needs_layout_passes=False),
    scratch_types=[
        pltpu.VMEM((EPT,), jnp.int32),
        pltpu.VMEM((NPAD,), jnp.float32),
        pltpu.VMEM((NS, RPT), jnp.float32),
        pltpu.VMEM((RPT,), jnp.float32),
        pltpu.VMEM_SHARED((NS, NPAD), jnp.float32),
    ],
)
def _deg_kernel(dst_hbm, deg_out, dst_v, hist_v, red_v, out_v, deg_sh):
    c = lax.axis_index("c")
    s = lax.axis_index("s")
    wid = s * NC + c

    zeros16 = jnp.zeros((16,), jnp.float32)
    ones16 = jnp.ones((16,), jnp.float32)

    def _zero(i, carry):
        hist_v[pl.ds(i * 16, 16)] = zeros16
        return carry

    lax.fori_loop(0, NPAD // 16, _zero, 0)

    pltpu.sync_copy(dst_hbm.at[wid], dst_v)

    def _hist(i, carry):
        for g in range(5):
            idx = dst_v[pl.ds((i * 5 + g) * 16, 16)]
            plsc.addupdate_scatter(hist_v, [idx], ones16)
        return carry

    lax.fori_loop(0, EPT // 80, _hist, 0)

    pltpu.sync_copy(hist_v, deg_sh.at[s])
    plsc.subcore_barrier()

    for r in range(NS):
        pltpu.sync_copy(deg_sh.at[r, pl.ds(s * RPT, RPT)], red_v.at[r])

    def _red(i, carry):
        acc = red_v[0, pl.ds(i * 16, 16)]
        for r in range(1, NS):
            acc = acc + red_v[r, pl.ds(i * 16, 16)]
        out_v[pl.ds(i * 16, 16)] = acc
        return carry

    lax.fori_loop(0, RPT // 16, _red, 0)

    pltpu.sync_copy(out_v, deg_out.at[c, pl.ds(s * RPT, RPT)])


@functools.partial(
    pl.kernel,
    out_type=jax.ShapeDtypeStruct((NC, NPAD, H), jnp.bfloat16),
    mesh=_mesh,
    compiler_params=pltpu.CompilerParams(
        needs_layout_passes=False, use_tc_tiling_on_sc=False),
    scratch_types=[
        pltpu.VMEM((GROUPS, GLEN), jnp.int32),
        pltpu.VMEM((GROUPS, GLEN), jnp.int32),
        [pltpu.VMEM((GLEN, H), jnp.bfloat16)] * 2,
        pltpu.VMEM_SHARED((NPAD, H), jnp.bfloat16),
        [pltpu.SemaphoreType.DMA] * 2,
    ],
)
def _msg_kernel(y_hbm, src_hbm, dst_hbm, acc_out,
                src_v, dst_v, bufs, acc_sh, gsems):
    c = lax.axis_index("c")
    s = lax.axis_index("s")
    wid = s * NC + c

    zeros32 = jnp.zeros((32,), jnp.bfloat16)

    def _zero(i, carry):
        for j in range(H // 32):
            bufs[0][i, pl.ds(j * 32, 32)] = zeros32
        return carry

    lax.fori_loop(0, RPT // 2, _zero, 0)
    for k in range(2):
        pltpu.sync_copy(bufs[0].at[pl.ds(0, RPT // 2)],
                        acc_sh.at[pl.ds(s * RPT + k * (RPT // 2), RPT // 2)])
    plsc.subcore_barrier()

    pltpu.sync_copy(src_hbm.at[wid], src_v)
    pltpu.sync_copy(dst_hbm.at[wid], dst_v)

    def _gather(g, k):
        pltpu.async_copy(y_hbm.at[src_v.at[g]], bufs[k], gsems[k])

    def _wait_gather(g, k):
        pltpu.make_async_copy(y_hbm.at[src_v.at[g]], bufs[k], gsems[k]).wait()

    _gather(0, 0)

    def _body(i, carry):
        for par in range(2):
            g = 2 * i + par
            k, kk = (par, 1 - par)

            @pl.when(g + 1 < GROUPS)
            def _():
                _gather(g + 1, kk)

            _wait_gather(g, k)
            pltpu.sync_copy(bufs[k], acc_sh.at[dst_v.at[g]], add=True)
        return carry

    lax.fori_loop(0, GROUPS // 2, _body, 0)
    plsc.subcore_barrier()

    pltpu.sync_copy(acc_sh.at[pl.ds(s * RPT, RPT)],
                    acc_out.at[c, pl.ds(s * RPT, RPT)])


def _tc1_body(x_ref, w_ref, d_ref, y_ref, dinv_ref):
    deg = d_ref[0] + d_ref[1] + 1.0
    dinv = lax.rsqrt(deg)
    xw = jnp.dot(x_ref[...], w_ref[...], preferred_element_type=jnp.float32)
    y_ref[...] = (xw * dinv).astype(jnp.bfloat16)
    dinv_ref[...] = dinv


_BLK1 = 2000


def _tc1(xb, Wb, deg3):
    return pl.pallas_call(
        _tc1_body,
        grid=(N // _BLK1,),
        in_specs=[
            pl.BlockSpec((_BLK1, D_IN), lambda i: (i, 0)),
            pl.BlockSpec((D_IN, H), lambda i: (0, 0)),
            pl.BlockSpec((NC, _BLK1, 1), lambda i: (0, i, 0)),
        ],
        out_specs=[
            pl.BlockSpec((_BLK1, H), lambda i: (i, 0)),
            pl.BlockSpec((_BLK1, 1), lambda i: (i, 0)),
        ],
        out_shape=[
            jax.ShapeDtypeStruct((N, H), jnp.bfloat16),
            jax.ShapeDtypeStruct((N, 1), jnp.float32),
        ],
    )(xb, Wb, deg3)


def _tc2_body(acc_ref, y_ref, dinv_ref, h_ref, b_ref,
              wri, wzi, wni, wrh, wzh, wnh,
              bri, bzi, bni, brh, bzh, bnh, o_ref):
    dinv = dinv_ref[...]
    f32sum = (acc_ref[0].astype(jnp.float32) + acc_ref[1].astype(jnp.float32)
              + y_ref[...].astype(jnp.float32))
    gcn = dinv * f32sum + b_ref[...]
    h = h_ref[...]
    f32 = jnp.float32
    i_r = jnp.dot(gcn, wri[...], preferred_element_type=f32) + bri[...]
    i_z = jnp.dot(gcn, wzi[...], preferred_element_type=f32) + bzi[...]
    i_n = jnp.dot(gcn, wni[...], preferred_element_type=f32) + bni[...]
    h_r = jnp.dot(h, wrh[...], preferred_element_type=f32) + brh[...]
    h_z = jnp.dot(h, wzh[...], preferred_element_type=f32) + bzh[...]
    h_n = jnp.dot(h, wnh[...], preferred_element_type=f32) + bnh[...]
    r = jax.nn.sigmoid(i_r + h_r)
    z = jax.nn.sigmoid(i_z + h_z)
    n = jnp.tanh(i_n + r * h_n)
    o_ref[...] = (1.0 - z) * n + z * h


_BLK2 = 2000


def _tc2(acc, y, dinv, h_mem, b, ws, bs):
    row = lambda i: (i, 0)
    full = lambda i: (0, 0)
    return pl.pallas_call(
        _tc2_body,
        grid=(N // _BLK2,),
        in_specs=[
            pl.BlockSpec((NC, _BLK2, H), lambda i: (0, i, 0)),
            pl.BlockSpec((_BLK2, H), row),
            pl.BlockSpec((_BLK2, 1), row),
            pl.BlockSpec((_BLK2, H), row),
            pl.BlockSpec((1, H), full),
        ]
        + [pl.BlockSpec((H, H), full)] * 6
        + [pl.BlockSpec((1, H), full)] * 6,
        out_specs=pl.BlockSpec((_BLK2, H), row),
        out_shape=jax.ShapeDtypeStruct((N, H), jnp.float32),
    )(acc, y, dinv, h_mem, b, *ws, *bs)


def kernel(x, edge_index, W, b, W_ih, W_hh, b_ih, b_hh, h_mem):
    ei = edge_index.astype(jnp.int32)
    srcp = ei[0].reshape(NW, GROUPS, GLEN)
    dstp = ei[1].reshape(NW, GROUPS, GLEN)
    dstf = ei[1].reshape(NW, EPT)

    deg2 = _deg_kernel(dstf)
    deg3 = deg2[:, :, None]

    y, dinv = _tc1(x.astype(jnp.bfloat16), W.astype(jnp.bfloat16),
                   deg3)

    acc = _msg_kernel(y, srcp, dstp)

    WiT = W_ih.T
    WhT = W_hh.T
    ws = (WiT[:, :H], WiT[:, H:2 * H], WiT[:, 2 * H:],
          WhT[:, :H], WhT[:, H:2 * H], WhT[:, 2 * H:])
    bs = (b_ih[None, :H], b_ih[None, H:2 * H], b_ih[None, 2 * H:],
          b_hh[None, :H], b_hh[None, H:2 * H], b_hh[None, 2 * H:])

    return _tc2(acc, y, dinv, h_mem, b[None, :], ws, bs)
